# baseline (speedup 1.0000x reference)
import functools

import jax
import jax.numpy as jnp
from jax import lax
from jax.experimental import pallas as pl
from jax.experimental.pallas import tpu as pltpu
from jax.experimental.pallas import tpu_sc as plsc

L = 16
NSUB = 16
PER = 3840
NPAD = NSUB * PER
CH = PER // L
SEG = PER + L
NCLS = 300
UNROLL = 4
HI0 = 0x7F800000
CAP1 = 512
CAP2 = 4096

_f32 = jnp.float32
_i32 = jnp.int32


def _softplus16(x):
    z = jnp.exp(-jnp.abs(x))
    w = z / (z + _f32(2.0))
    w2 = w * w
    p = _f32(1.0 / 13.0)
    p = _f32(1.0 / 11.0) + w2 * p
    p = _f32(1.0 / 9.0) + w2 * p
    p = _f32(1.0 / 7.0) + w2 * p
    p = _f32(1.0 / 5.0) + w2 * p
    p = _f32(1.0 / 3.0) + w2 * p
    p = _f32(1.0) + w2 * p
    return jnp.maximum(x, _f32(0.0)) + _f32(2.0) * w * p


@functools.cache
def _build():
    mesh = plsc.VectorSubcoreMesh(core_axis_name="c", subcore_axis_name="s")

    @functools.partial(
        pl.kernel,
        out_type=jax.ShapeDtypeStruct((L,), _f32),
        mesh=mesh,
        compiler_params=pltpu.CompilerParams(needs_layout_passes=False),
        scratch_types=[
            pltpu.VMEM((PER,), _f32),
            pltpu.VMEM((PER,), _f32),
            pltpu.VMEM((PER,), _i32),
            pltpu.VMEM((SEG,), _f32),
            pltpu.VMEM((NSUB * SEG + L,), _f32),
            pltpu.VMEM((NSUB * L,), _f32),
            pltpu.VMEM((L,), _f32),
            pltpu.VMEM((L,), _f32),
            pltpu.SMEM((4,), _i32),
            pltpu.VMEM_SHARED((NSUB * SEG,), _f32),
            pltpu.VMEM_SHARED((NSUB * L,), _f32),
            pltpu.SemaphoreType.DMA,
            pltpu.SemaphoreType.DMA,
            pltpu.SemaphoreType.DMA,
        ],
    )
    def k(l0_hbm, l1_hbm, lab_hbm, out_hbm,
          l0_v, l1_v, lab_v, bgbuf, dense, meta_fg_v, stage_fg, outbuf,
          counters, sh_bg, sh_fg, sem0, sem1, sem2):
        cid = lax.axis_index("c")
        sid = lax.axis_index("s")

        @pl.when(cid == 0)
        def _core0():
            zf = jnp.zeros((L,), _f32)
            zi = jnp.zeros((L,), _i32)
            lane = lax.broadcasted_iota(_i32, (L,), 0)

            base = sid * PER
            c0 = pltpu.async_copy(l0_hbm.at[pl.ds(base, PER)], l0_v, sem0)
            c1 = pltpu.async_copy(l1_hbm.at[pl.ds(base, PER)], l1_v, sem1)
            c2 = pltpu.async_copy(lab_hbm.at[pl.ds(base, PER)], lab_v, sem2)

            @pl.when(sid == 0)
            def _init():
                counters[_i32(0)] = _i32(0)
                counters[_i32(1)] = _i32(0)
                counters[_i32(2)] = _i32(0)

            c0.wait()
            c1.wait()
            c2.wait()
            plsc.subcore_barrier()

            def body(i, carry):
                off_v, fg_acc, nfg_acc = carry
                for u in range(UNROLL):
                    sl = pl.ds((i * UNROLL + u) * L, L)
                    x0 = l0_v[sl]
                    x1 = l1_v[sl]
                    lb = lab_v[sl]
                    dd = x1 - x0
                    is_fg = lb == 1
                    is_bg = lb == 0
                    loss = _softplus16(jnp.where(is_fg, -dd, dd))
                    fg_acc = fg_acc + jnp.where(is_fg, loss, _f32(0.0))
                    nfg_acc = nfg_acc + jnp.where(is_fg, _i32(1), _i32(0))
                    bg_i = jnp.where(is_bg, _i32(1), _i32(0))
                    pos = off_v + lax.cumsum(bg_i, axis=0) - _i32(1)
                    plsc.store_scatter(bgbuf, [pos], loss, mask=is_bg)
                    off_v = off_v + plsc.all_reduce_population_count(is_bg)
                return off_v, fg_acc, nfg_acc

            off_v, fg_acc, nfg_acc = lax.fori_loop(
                _i32(0), _i32(CH // UNROLL), body, (zi, zf, zi))
            off = jnp.max(off_v)
            plsc.store_scatter(bgbuf, [off + lane],
                               jnp.full((L,), -jnp.inf, _f32))

            nch = lax.shift_right_logical(off + _i32(L - 1), _i32(4))
            words = nch * _i32(L)
            gbase = plsc.fetch_and_add(counters.at[_i32(0)], words, subcore_id=_i32(0))
            plsc.fetch_and_add(counters.at[_i32(1)], off, subcore_id=_i32(0))
            nfg_me = jnp.sum(nfg_acc, dtype=_i32)
            plsc.fetch_and_add(counters.at[_i32(2)], nfg_me, subcore_id=_i32(0))

            def cp(j, _):
                pltpu.sync_copy(bgbuf.at[pl.ds(j * L, L)],
                                sh_bg.at[pl.ds(pl.multiple_of(gbase + j * L, L), L)])
                return _

            lax.fori_loop(_i32(0), nch, cp, _i32(0))
            stage_fg[...] = fg_acc
            pltpu.sync_copy(stage_fg, sh_fg.at[pl.ds(sid * L, L)])
            plsc.subcore_barrier()

            @pl.when(sid == 0)
            def _merge():
                gw = counters[_i32(0)]
                n_bg = counters[_i32(1)]
                n_fg = counters[_i32(2)]
                G = lax.shift_right_logical(gw, _i32(4))
                pltpu.sync_copy(sh_fg, meta_fg_v)

                @pl.when(gw <= CAP1)
                def _small():
                    pltpu.sync_copy(sh_bg.at[pl.ds(0, CAP1)],
                                    dense.at[pl.ds(0, CAP1)])

                @pl.when((gw > CAP1) & (gw <= CAP2))
                def _mid():
                    pltpu.sync_copy(sh_bg.at[pl.ds(0, CAP2)],
                                    dense.at[pl.ds(0, CAP2)])

                @pl.when(gw > CAP2)
                def _full():
                    pltpu.sync_copy(sh_bg, dense.at[pl.ds(0, NSUB * SEG)])

                def red(w_, fg_v):
                    return fg_v + meta_fg_v[pl.ds(w_ * L, L)]

                fg_v = lax.fori_loop(_i32(0), _i32(NSUB), red, zf)
                fg_sum = jnp.sum(fg_v)
                K = _i32(NCLS) - n_fg
                ninf = jnp.full((L,), -jnp.inf, _f32)

                def _finish(c_gt, s_gt, tv):
                    t_s = jnp.max(tv)
                    bg_main = s_gt + (K - c_gt).astype(_f32) * t_s
                    bg_sum = jnp.where(
                        K <= _i32(0), _f32(0.0),
                        jnp.where(K > n_bg, _f32(-jnp.inf), bg_main))
                    outbuf[...] = (zf + (fg_sum + bg_sum)) / (zf + _f32(NCLS))
                    pltpu.sync_copy(outbuf, out_hbm)


                @pl.when(G <= _i32(L))
                def _reg_search():
                    vs = [jnp.where(_i32(c) < G, dense[pl.ds(c * L, L)],
                                    ninf) for c in range(L)]

                    def bs(_, carry):
                        lo, hi = carry
                        mid = lo + lax.shift_right_logical(hi - lo, _i32(1))
                        tv = plsc.bitcast(zi + mid, _f32)
                        acc = zi
                        for c in range(L):
                            acc = acc + jnp.where(vs[c] >= tv,
                                                  _i32(1), _i32(0))
                        pred = jnp.sum(acc, dtype=_i32) >= K
                        return (jnp.where(pred, mid, lo),
                                jnp.where(pred, hi, mid))

                    lo, _hi = lax.fori_loop(_i32(0), _i32(31), bs,
                                            (_i32(0), _i32(HI0)))
                    tv = plsc.bitcast(zi + lo, _f32)
                    cv = zi
                    sv = zf
                    for c in range(L):
                        m = vs[c] > tv
                        cv = cv + jnp.where(m, _i32(1), _i32(0))
                        sv = sv + jnp.where(m, vs[c], _f32(0.0))
                    _finish(jnp.sum(cv, dtype=_i32), jnp.sum(sv), tv)

                @pl.when(G > _i32(L))
                def _loop_search():
                    dense[pl.ds(G * L, L)] = ninf
                    G2 = lax.shift_right_logical(G + _i32(1), _i32(1))

                    def bs(_, carry):
                        lo, hi = carry
                        mid = lo + lax.shift_right_logical(hi - lo, _i32(1))
                        tv = plsc.bitcast(zi + mid, _f32)

                        def cb(j, acc):
                            va = dense[pl.ds(j * (2 * L), L)]
                            vb = dense[pl.ds(j * (2 * L) + L, L)]
                            return (acc
                                    + jnp.where(va >= tv, _i32(1), _i32(0))
                                    + jnp.where(vb >= tv, _i32(1), _i32(0)))

                        c = jnp.sum(lax.fori_loop(_i32(0), G2, cb, zi),
                                    dtype=_i32)
                        pred = c >= K
                        return (jnp.where(pred, mid, lo),
                                jnp.where(pred, hi, mid))

                    lo, _hi = lax.fori_loop(_i32(0), _i32(31), bs,
                                            (_i32(0), _i32(HI0)))
                    tv = plsc.bitcast(zi + lo, _f32)

                    def fin(j, carry):
                        cv, sv = carry
                        va = dense[pl.ds(j * (2 * L), L)]
                        vb = dense[pl.ds(j * (2 * L) + L, L)]
                        ma = va > tv
                        mb = vb > tv
                        return (cv + jnp.where(ma, _i32(1), _i32(0))
                                + jnp.where(mb, _i32(1), _i32(0)),
                                sv + jnp.where(ma, va, _f32(0.0))
                                + jnp.where(mb, vb, _f32(0.0)))

                    cv, sv = lax.fori_loop(_i32(0), G2, fin, (zi, zf))
                    _finish(jnp.sum(cv, dtype=_i32), jnp.sum(sv), tv)

    return k


def kernel(输入, 标签):
    logits = 输入[0]
    labels = 标签[0, 0].astype(_i32)
    n = logits.shape[0]
    pad = NPAD - n
    l0 = jnp.concatenate([logits[:, 0], jnp.zeros((pad,), _f32)])
    l1 = jnp.concatenate([logits[:, 1], jnp.zeros((pad,), _f32)])
    lab = jnp.concatenate([labels, jnp.full((pad,), 2, _i32)])
    out = _build()(l0, l1, lab)
    return out[0]


# speedup vs baseline: 1.2316x; 1.2316x over previous
import functools

import jax
import jax.numpy as jnp
from jax import lax
from jax.experimental import pallas as pl
from jax.experimental.pallas import tpu as pltpu
from jax.experimental.pallas import tpu_sc as plsc

L = 16
NSUB = 16
PER = 3840
NPAD = NSUB * PER
CH = PER // L
SEG = PER + L
NCLS = 300
UNROLL = 4
HI0 = 0x7F800000
CAP1 = 512
CAP2 = 4096

_f32 = jnp.float32
_i32 = jnp.int32


def _softplus16(x):
    z = jnp.exp(-jnp.abs(x))
    w = z / (z + _f32(2.0))
    w2 = w * w
    p = _f32(1.0 / 13.0)
    p = _f32(1.0 / 11.0) + w2 * p
    p = _f32(1.0 / 9.0) + w2 * p
    p = _f32(1.0 / 7.0) + w2 * p
    p = _f32(1.0 / 5.0) + w2 * p
    p = _f32(1.0 / 3.0) + w2 * p
    p = _f32(1.0) + w2 * p
    return jnp.maximum(x, _f32(0.0)) + _f32(2.0) * w * p


@functools.cache
def _build():
    mesh = plsc.VectorSubcoreMesh(core_axis_name="c", subcore_axis_name="s")

    @functools.partial(
        pl.kernel,
        out_type=jax.ShapeDtypeStruct((L,), _f32),
        mesh=mesh,
        compiler_params=pltpu.CompilerParams(needs_layout_passes=False),
        scratch_types=[
            pltpu.VMEM((PER,), _f32),
            pltpu.VMEM((PER,), _f32),
            pltpu.VMEM((PER,), _i32),
            pltpu.VMEM((SEG,), _f32),
            pltpu.VMEM((SEG,), _f32),
            pltpu.VMEM((SEG,), _i32),
            pltpu.VMEM((NSUB * SEG + L,), _f32),
            pltpu.VMEM((NSUB * L,), _f32),
            pltpu.VMEM((L,), _f32),
            pltpu.VMEM((L,), _f32),
            pltpu.SMEM((4,), _i32),
            pltpu.VMEM_SHARED((NSUB * SEG,), _f32),
            pltpu.VMEM_SHARED((NSUB * L,), _f32),
            pltpu.SemaphoreType.DMA,
            pltpu.SemaphoreType.DMA,
            pltpu.SemaphoreType.DMA,
        ],
    )
    def k(l0_hbm, l1_hbm, lab_hbm, out_hbm,
          l0_v, l1_v, lab_v, bgbuf, relbuf_d, relbuf_f, dense,
          meta_fg_v, stage_fg, outbuf,
          counters, sh_bg, sh_fg, sem0, sem1, sem2):
        cid = lax.axis_index("c")
        sid = lax.axis_index("s")

        @pl.when(cid == 0)
        def _core0():
            zf = jnp.zeros((L,), _f32)
            zi = jnp.zeros((L,), _i32)
            lane = lax.broadcasted_iota(_i32, (L,), 0)

            base = sid * PER
            c0 = pltpu.async_copy(l0_hbm.at[pl.ds(base, PER)], l0_v, sem0)
            c1 = pltpu.async_copy(l1_hbm.at[pl.ds(base, PER)], l1_v, sem1)
            c2 = pltpu.async_copy(lab_hbm.at[pl.ds(base, PER)], lab_v, sem2)

            @pl.when(sid == 0)
            def _init():
                counters[_i32(0)] = _i32(0)
                counters[_i32(1)] = _i32(0)
                counters[_i32(2)] = _i32(0)

            c0.wait()
            c1.wait()
            c2.wait()
            plsc.subcore_barrier()

            def body(i, carry):
                offr_v = carry
                for u in range(UNROLL):
                    sl = pl.ds((i * UNROLL + u) * L, L)
                    x0 = l0_v[sl]
                    x1 = l1_v[sl]
                    lb = lab_v[sl]
                    dd = x1 - x0
                    sd = jnp.where(lb == 1, -dd, dd)
                    rel = lb < 2
                    rel_i = jnp.where(rel, _i32(1), _i32(0))
                    pos = offr_v + lax.cumsum(rel_i, axis=0) - _i32(1)
                    plsc.store_scatter(relbuf_d, [pos], sd, mask=rel)
                    plsc.store_scatter(relbuf_f, [pos], lb, mask=rel)
                    offr_v = offr_v + plsc.all_reduce_population_count(rel)
                return offr_v

            offr_v = lax.fori_loop(
                _i32(0), _i32(CH // UNROLL), body, zi)
            offr = jnp.max(offr_v)
            plsc.store_scatter(relbuf_f, [offr + lane], zi + _i32(2))
            crel = lax.shift_right_logical(offr + _i32(L - 1), _i32(4))

            def sp(j, carry):
                off_v, fg_acc, nfg_acc = carry
                sl = pl.ds(j * L, L)
                sd = relbuf_d[sl]
                f = relbuf_f[sl]
                loss = _softplus16(sd)
                is_fg = f == 1
                is_bg = f == 0
                fg_acc = fg_acc + jnp.where(is_fg, loss, _f32(0.0))
                nfg_acc = nfg_acc + jnp.where(is_fg, _i32(1), _i32(0))
                bg_i = jnp.where(is_bg, _i32(1), _i32(0))
                pos = off_v + lax.cumsum(bg_i, axis=0) - _i32(1)
                plsc.store_scatter(bgbuf, [pos], loss, mask=is_bg)
                off_v = off_v + plsc.all_reduce_population_count(is_bg)
                return off_v, fg_acc, nfg_acc

            off_v, fg_acc, nfg_acc = lax.fori_loop(
                _i32(0), crel, sp, (zi, zf, zi))
            off = jnp.max(off_v)
            plsc.store_scatter(bgbuf, [off + lane],
                               jnp.full((L,), -jnp.inf, _f32))

            nch = lax.shift_right_logical(off + _i32(L - 1), _i32(4))
            words = nch * _i32(L)
            gbase = plsc.fetch_and_add(counters.at[_i32(0)], words, subcore_id=_i32(0))
            plsc.fetch_and_add(counters.at[_i32(1)], off, subcore_id=_i32(0))
            nfg_me = jnp.sum(nfg_acc, dtype=_i32)
            plsc.fetch_and_add(counters.at[_i32(2)], nfg_me, subcore_id=_i32(0))

            def cp(j, _):
                pltpu.sync_copy(bgbuf.at[pl.ds(j * L, L)],
                                sh_bg.at[pl.ds(pl.multiple_of(gbase + j * L, L), L)])
                return _

            lax.fori_loop(_i32(0), nch, cp, _i32(0))
            stage_fg[...] = fg_acc
            pltpu.sync_copy(stage_fg, sh_fg.at[pl.ds(sid * L, L)])
            plsc.subcore_barrier()

            @pl.when(sid == 0)
            def _merge():
                gw = counters[_i32(0)]
                n_bg = counters[_i32(1)]
                n_fg = counters[_i32(2)]
                G = lax.shift_right_logical(gw, _i32(4))
                pltpu.sync_copy(sh_fg, meta_fg_v)

                @pl.when(gw <= CAP1)
                def _small():
                    pltpu.sync_copy(sh_bg.at[pl.ds(0, CAP1)],
                                    dense.at[pl.ds(0, CAP1)])

                @pl.when((gw > CAP1) & (gw <= CAP2))
                def _mid():
                    pltpu.sync_copy(sh_bg.at[pl.ds(0, CAP2)],
                                    dense.at[pl.ds(0, CAP2)])

                @pl.when(gw > CAP2)
                def _full():
                    pltpu.sync_copy(sh_bg, dense.at[pl.ds(0, NSUB * SEG)])

                def red(w_, fg_v):
                    return fg_v + meta_fg_v[pl.ds(w_ * L, L)]

                fg_v = lax.fori_loop(_i32(0), _i32(NSUB), red, zf)
                fg_sum = jnp.sum(fg_v)
                K = _i32(NCLS) - n_fg
                ninf = jnp.full((L,), -jnp.inf, _f32)

                def _finish(c_gt, s_gt, tv):
                    t_s = jnp.max(tv)
                    bg_main = s_gt + (K - c_gt).astype(_f32) * t_s
                    bg_sum = jnp.where(
                        K <= _i32(0), _f32(0.0),
                        jnp.where(K > n_bg, _f32(-jnp.inf), bg_main))
                    outbuf[...] = (zf + (fg_sum + bg_sum)) / (zf + _f32(NCLS))
                    pltpu.sync_copy(outbuf, out_hbm)


                @pl.when(G <= _i32(L))
                def _reg_search():
                    vs = [jnp.where(_i32(c) < G, dense[pl.ds(c * L, L)],
                                    ninf) for c in range(L)]

                    def bs(_, carry):
                        lo, hi = carry
                        mid = lo + lax.shift_right_logical(hi - lo, _i32(1))
                        tv = plsc.bitcast(zi + mid, _f32)
                        acc = zi
                        for c in range(L):
                            acc = acc + jnp.where(vs[c] >= tv,
                                                  _i32(1), _i32(0))
                        pred = jnp.sum(acc, dtype=_i32) >= K
                        return (jnp.where(pred, mid, lo),
                                jnp.where(pred, hi, mid))

                    lo, _hi = lax.fori_loop(_i32(0), _i32(31), bs,
                                            (_i32(0), _i32(HI0)))
                    tv = plsc.bitcast(zi + lo, _f32)
                    cv = zi
                    sv = zf
                    for c in range(L):
                        m = vs[c] > tv
                        cv = cv + jnp.where(m, _i32(1), _i32(0))
                        sv = sv + jnp.where(m, vs[c], _f32(0.0))
                    _finish(jnp.sum(cv, dtype=_i32), jnp.sum(sv), tv)

                @pl.when(G > _i32(L))
                def _loop_search():
                    dense[pl.ds(G * L, L)] = ninf
                    G2 = lax.shift_right_logical(G + _i32(1), _i32(1))

                    def bs(_, carry):
                        lo, hi = carry
                        mid = lo + lax.shift_right_logical(hi - lo, _i32(1))
                        tv = plsc.bitcast(zi + mid, _f32)

                        def cb(j, acc):
                            va = dense[pl.ds(j * (2 * L), L)]
                            vb = dense[pl.ds(j * (2 * L) + L, L)]
                            return (acc
                                    + jnp.where(va >= tv, _i32(1), _i32(0))
                                    + jnp.where(vb >= tv, _i32(1), _i32(0)))

                        c = jnp.sum(lax.fori_loop(_i32(0), G2, cb, zi),
                                    dtype=_i32)
                        pred = c >= K
                        return (jnp.where(pred, mid, lo),
                                jnp.where(pred, hi, mid))

                    lo, _hi = lax.fori_loop(_i32(0), _i32(31), bs,
                                            (_i32(0), _i32(HI0)))
                    tv = plsc.bitcast(zi + lo, _f32)

                    def fin(j, carry):
                        cv, sv = carry
                        va = dense[pl.ds(j * (2 * L), L)]
                        vb = dense[pl.ds(j * (2 * L) + L, L)]
                        ma = va > tv
                        mb = vb > tv
                        return (cv + jnp.where(ma, _i32(1), _i32(0))
                                + jnp.where(mb, _i32(1), _i32(0)),
                                sv + jnp.where(ma, va, _f32(0.0))
                                + jnp.where(mb, vb, _f32(0.0)))

                    cv, sv = lax.fori_loop(_i32(0), G2, fin, (zi, zf))
                    _finish(jnp.sum(cv, dtype=_i32), jnp.sum(sv), tv)

    return k


def kernel(输入, 标签):
    logits = 输入[0]
    labels = 标签[0, 0].astype(_i32)
    n = logits.shape[0]
    pad = NPAD - n
    l0 = jnp.concatenate([logits[:, 0], jnp.zeros((pad,), _f32)])
    l1 = jnp.concatenate([logits[:, 1], jnp.zeros((pad,), _f32)])
    lab = jnp.concatenate([labels, jnp.full((pad,), 2, _i32)])
    out = _build()(l0, l1, lab)
    return out[0]


# speedup vs baseline: 1.2592x; 1.0224x over previous
import functools

import jax
import jax.numpy as jnp
from jax import lax
from jax.experimental import pallas as pl
from jax.experimental.pallas import tpu as pltpu
from jax.experimental.pallas import tpu_sc as plsc

L = 16
NSUB = 16
N = 60000
STRIDE = 3744
PER = 3840
CH = PER // L
SEG = PER + L
NCLS = 300
UNROLL = 8
HI0 = 0x7F800000
CAP1 = 512
CAP2 = 4096

_f32 = jnp.float32
_i32 = jnp.int32


def _softplus16(x):
    z = jnp.exp(-jnp.abs(x))
    w = z / (z + _f32(2.0))
    w2 = w * w
    p = _f32(1.0 / 13.0)
    p = _f32(1.0 / 11.0) + w2 * p
    p = _f32(1.0 / 9.0) + w2 * p
    p = _f32(1.0 / 7.0) + w2 * p
    p = _f32(1.0 / 5.0) + w2 * p
    p = _f32(1.0 / 3.0) + w2 * p
    p = _f32(1.0) + w2 * p
    return jnp.maximum(x, _f32(0.0)) + _f32(2.0) * w * p


@functools.cache
def _build():
    mesh = plsc.VectorSubcoreMesh(core_axis_name="c", subcore_axis_name="s")

    @functools.partial(
        pl.kernel,
        out_type=jax.ShapeDtypeStruct((L,), _f32),
        mesh=mesh,
        compiler_params=pltpu.CompilerParams(needs_layout_passes=False),
        scratch_types=[
            pltpu.VMEM((PER,), _f32),
            pltpu.VMEM((PER,), _f32),
            pltpu.VMEM((PER,), _i32),
            pltpu.VMEM((SEG,), _f32),
            pltpu.VMEM((SEG,), _f32),
            pltpu.VMEM((SEG,), _i32),
            pltpu.VMEM((NSUB * SEG + L,), _f32),
            pltpu.VMEM((NSUB * L,), _f32),
            pltpu.VMEM((L,), _f32),
            pltpu.VMEM((L,), _f32),
            pltpu.SMEM((4,), _i32),
            pltpu.VMEM_SHARED((NSUB * SEG,), _f32),
            pltpu.VMEM_SHARED((NSUB * L,), _f32),
            pltpu.SemaphoreType.DMA,
            pltpu.SemaphoreType.DMA,
            pltpu.SemaphoreType.DMA,
        ],
    )
    def k(l0_hbm, l1_hbm, lab_hbm, out_hbm,
          l0_v, l1_v, lab_v, bgbuf, relbuf_d, relbuf_f, dense,
          meta_fg_v, stage_fg, outbuf,
          counters, sh_bg, sh_fg, sem0, sem1, sem2):
        cid = lax.axis_index("c")
        sid = lax.axis_index("s")

        @pl.when(cid == 0)
        def _core0():
            zf = jnp.zeros((L,), _f32)
            zi = jnp.zeros((L,), _i32)
            lane = lax.broadcasted_iota(_i32, (L,), 0)

            base = sid * STRIDE
            rlim_v = (zi + jnp.where(sid == NSUB - 1, _i32(N),
                                     (sid + _i32(1)) * _i32(STRIDE))) - base
            c0 = pltpu.async_copy(l0_hbm.at[pl.ds(base, PER)], l0_v, sem0)
            c1 = pltpu.async_copy(l1_hbm.at[pl.ds(base, PER)], l1_v, sem1)
            c2 = pltpu.async_copy(lab_hbm.at[pl.ds(base, PER)], lab_v, sem2)

            @pl.when(sid == 0)
            def _init():
                counters[_i32(0)] = _i32(0)
                counters[_i32(1)] = _i32(0)
                counters[_i32(2)] = _i32(0)

            c0.wait()
            c1.wait()
            c2.wait()
            plsc.subcore_barrier()

            def body(i, carry):
                offr_v = carry
                for u in range(UNROLL):
                    sl = pl.ds((i * UNROLL + u) * L, L)
                    x0 = l0_v[sl]
                    x1 = l1_v[sl]
                    lb = lab_v[sl]
                    dd = x1 - x0
                    sd = jnp.where(lb == 1, -dd, dd)
                    rel = (lb < 2) & ((lane + (i * UNROLL + u) * L)
                                      < rlim_v)
                    rel_i = jnp.where(rel, _i32(1), _i32(0))
                    pos = offr_v + lax.cumsum(rel_i, axis=0) - _i32(1)
                    plsc.store_scatter(relbuf_d, [pos], sd, mask=rel)
                    plsc.store_scatter(relbuf_f, [pos], lb, mask=rel)
                    offr_v = offr_v + plsc.all_reduce_population_count(rel)
                return offr_v

            offr_v = lax.fori_loop(
                _i32(0), _i32(CH // UNROLL), body, zi)
            offr = jnp.max(offr_v)
            plsc.store_scatter(relbuf_f, [offr + lane], zi + _i32(2))
            crel = lax.shift_right_logical(offr + _i32(L - 1), _i32(4))

            def sp(j, carry):
                off_v, fg_acc, nfg_acc = carry
                sl = pl.ds(j * L, L)
                sd = relbuf_d[sl]
                f = relbuf_f[sl]
                loss = _softplus16(sd)
                is_fg = f == 1
                is_bg = f == 0
                fg_acc = fg_acc + jnp.where(is_fg, loss, _f32(0.0))
                nfg_acc = nfg_acc + jnp.where(is_fg, _i32(1), _i32(0))
                bg_i = jnp.where(is_bg, _i32(1), _i32(0))
                pos = off_v + lax.cumsum(bg_i, axis=0) - _i32(1)
                plsc.store_scatter(bgbuf, [pos], loss, mask=is_bg)
                off_v = off_v + plsc.all_reduce_population_count(is_bg)
                return off_v, fg_acc, nfg_acc

            off_v, fg_acc, nfg_acc = lax.fori_loop(
                _i32(0), crel, sp, (zi, zf, zi))
            off = jnp.max(off_v)
            plsc.store_scatter(bgbuf, [off + lane],
                               jnp.full((L,), -jnp.inf, _f32))

            nch = lax.shift_right_logical(off + _i32(L - 1), _i32(4))
            words = nch * _i32(L)
            gbase = plsc.fetch_and_add(counters.at[_i32(0)], words, subcore_id=_i32(0))
            plsc.fetch_and_add(counters.at[_i32(1)], off, subcore_id=_i32(0))
            nfg_me = jnp.sum(nfg_acc, dtype=_i32)
            plsc.fetch_and_add(counters.at[_i32(2)], nfg_me, subcore_id=_i32(0))

            def cp(j, _):
                pltpu.sync_copy(bgbuf.at[pl.ds(j * L, L)],
                                sh_bg.at[pl.ds(pl.multiple_of(gbase + j * L, L), L)])
                return _

            lax.fori_loop(_i32(0), nch, cp, _i32(0))
            stage_fg[...] = fg_acc
            pltpu.sync_copy(stage_fg, sh_fg.at[pl.ds(sid * L, L)])
            plsc.subcore_barrier()

            @pl.when(sid == 0)
            def _merge():
                gw = counters[_i32(0)]
                n_bg = counters[_i32(1)]
                n_fg = counters[_i32(2)]
                G = lax.shift_right_logical(gw, _i32(4))
                pltpu.sync_copy(sh_fg, meta_fg_v)

                @pl.when(gw <= CAP1)
                def _small():
                    pltpu.sync_copy(sh_bg.at[pl.ds(0, CAP1)],
                                    dense.at[pl.ds(0, CAP1)])

                @pl.when((gw > CAP1) & (gw <= CAP2))
                def _mid():
                    pltpu.sync_copy(sh_bg.at[pl.ds(0, CAP2)],
                                    dense.at[pl.ds(0, CAP2)])

                @pl.when(gw > CAP2)
                def _full():
                    pltpu.sync_copy(sh_bg, dense.at[pl.ds(0, NSUB * SEG)])

                def red(w_, fg_v):
                    return fg_v + meta_fg_v[pl.ds(w_ * L, L)]

                fg_v = lax.fori_loop(_i32(0), _i32(NSUB), red, zf)
                fg_sum = jnp.sum(fg_v)
                K = _i32(NCLS) - n_fg
                ninf = jnp.full((L,), -jnp.inf, _f32)

                def _finish(c_gt, s_gt, tv):
                    t_s = jnp.max(tv)
                    bg_main = s_gt + (K - c_gt).astype(_f32) * t_s
                    bg_sum = jnp.where(
                        K <= _i32(0), _f32(0.0),
                        jnp.where(K > n_bg, _f32(-jnp.inf), bg_main))
                    outbuf[...] = (zf + (fg_sum + bg_sum)) / (zf + _f32(NCLS))
                    pltpu.sync_copy(outbuf, out_hbm)


                @pl.when(G <= _i32(L))
                def _reg_search():
                    vs = [jnp.where(_i32(c) < G, dense[pl.ds(c * L, L)],
                                    ninf) for c in range(L)]

                    def bs(_, carry):
                        lo, hi = carry
                        mid = lo + lax.shift_right_logical(hi - lo, _i32(1))
                        tv = plsc.bitcast(zi + mid, _f32)
                        acc = zi
                        for c in range(L):
                            acc = acc + jnp.where(vs[c] >= tv,
                                                  _i32(1), _i32(0))
                        pred = jnp.sum(acc, dtype=_i32) >= K
                        return (jnp.where(pred, mid, lo),
                                jnp.where(pred, hi, mid))

                    lo, _hi = lax.fori_loop(_i32(0), _i32(31), bs,
                                            (_i32(0), _i32(HI0)))
                    tv = plsc.bitcast(zi + lo, _f32)
                    cv = zi
                    sv = zf
                    for c in range(L):
                        m = vs[c] > tv
                        cv = cv + jnp.where(m, _i32(1), _i32(0))
                        sv = sv + jnp.where(m, vs[c], _f32(0.0))
                    _finish(jnp.sum(cv, dtype=_i32), jnp.sum(sv), tv)

                @pl.when(G > _i32(L))
                def _loop_search():
                    dense[pl.ds(G * L, L)] = ninf
                    G2 = lax.shift_right_logical(G + _i32(1), _i32(1))

                    def bs(_, carry):
                        lo, hi = carry
                        mid = lo + lax.shift_right_logical(hi - lo, _i32(1))
                        tv = plsc.bitcast(zi + mid, _f32)

                        def cb(j, acc):
                            va = dense[pl.ds(j * (2 * L), L)]
                            vb = dense[pl.ds(j * (2 * L) + L, L)]
                            return (acc
                                    + jnp.where(va >= tv, _i32(1), _i32(0))
                                    + jnp.where(vb >= tv, _i32(1), _i32(0)))

                        c = jnp.sum(lax.fori_loop(_i32(0), G2, cb, zi),
                                    dtype=_i32)
                        pred = c >= K
                        return (jnp.where(pred, mid, lo),
                                jnp.where(pred, hi, mid))

                    lo, _hi = lax.fori_loop(_i32(0), _i32(31), bs,
                                            (_i32(0), _i32(HI0)))
                    tv = plsc.bitcast(zi + lo, _f32)

                    def fin(j, carry):
                        cv, sv = carry
                        va = dense[pl.ds(j * (2 * L), L)]
                        vb = dense[pl.ds(j * (2 * L) + L, L)]
                        ma = va > tv
                        mb = vb > tv
                        return (cv + jnp.where(ma, _i32(1), _i32(0))
                                + jnp.where(mb, _i32(1), _i32(0)),
                                sv + jnp.where(ma, va, _f32(0.0))
                                + jnp.where(mb, vb, _f32(0.0)))

                    cv, sv = lax.fori_loop(_i32(0), G2, fin, (zi, zf))
                    _finish(jnp.sum(cv, dtype=_i32), jnp.sum(sv), tv)

    return k


def kernel(输入, 标签):
    logits = 输入[0]
    lab = 标签[0, 0].astype(_i32)
    out = _build()(logits[:, 0], logits[:, 1], lab)
    return out[0]


# speedup vs baseline: 1.3254x; 1.0526x over previous
import functools

import jax
import jax.numpy as jnp
from jax import lax
from jax.experimental import pallas as pl
from jax.experimental.pallas import tpu as pltpu
from jax.experimental.pallas import tpu_sc as plsc

L = 16
NSUB = 16
N = 60000
STRIDE = 3744
PER = 3840
CH = PER // L
SEG = PER + L
NCLS = 300
UNROLL = 8
HI0 = 0x7F800000
CAP1 = 512
CAP2 = 4096

_f32 = jnp.float32
_i32 = jnp.int32


def _softplus16(x):
    z = jnp.exp(-jnp.abs(x))
    w = z / (z + _f32(2.0))
    w2 = w * w
    p = _f32(1.0 / 13.0)
    p = _f32(1.0 / 11.0) + w2 * p
    p = _f32(1.0 / 9.0) + w2 * p
    p = _f32(1.0 / 7.0) + w2 * p
    p = _f32(1.0 / 5.0) + w2 * p
    p = _f32(1.0 / 3.0) + w2 * p
    p = _f32(1.0) + w2 * p
    return jnp.maximum(x, _f32(0.0)) + _f32(2.0) * w * p


@functools.cache
def _build():
    mesh = plsc.VectorSubcoreMesh(core_axis_name="c", subcore_axis_name="s", num_cores=1)

    @functools.partial(
        pl.kernel,
        out_type=jax.ShapeDtypeStruct((L,), _f32),
        mesh=mesh,
        compiler_params=pltpu.CompilerParams(needs_layout_passes=False),
        scratch_types=[
            pltpu.VMEM((PER,), _f32),
            pltpu.VMEM((PER,), _f32),
            pltpu.VMEM((PER,), _i32),
            pltpu.VMEM((SEG,), _f32),
            pltpu.VMEM((SEG,), _f32),
            pltpu.VMEM((SEG,), _i32),
            pltpu.VMEM((NSUB * SEG + L,), _f32),
            pltpu.VMEM((NSUB * L,), _f32),
            pltpu.VMEM((L,), _f32),
            pltpu.VMEM((L,), _f32),
            pltpu.SMEM((4,), _i32),
            pltpu.VMEM_SHARED((NSUB * SEG,), _f32),
            pltpu.VMEM_SHARED((NSUB * L,), _f32),
            pltpu.SemaphoreType.DMA,
            pltpu.SemaphoreType.DMA,
            pltpu.SemaphoreType.DMA,
        ],
    )
    def k(l0_hbm, l1_hbm, lab_hbm, out_hbm,
          l0_v, l1_v, lab_v, bgbuf, relbuf_d, relbuf_f, dense,
          meta_fg_v, stage_fg, outbuf,
          counters, sh_bg, sh_fg, sem0, sem1, sem2):
        cid = lax.axis_index("c")
        sid = lax.axis_index("s")

        @pl.when(cid == 0)
        def _core0():
            zf = jnp.zeros((L,), _f32)
            zi = jnp.zeros((L,), _i32)
            lane = lax.broadcasted_iota(_i32, (L,), 0)

            base = sid * STRIDE
            rlim_v = (zi + jnp.where(sid == NSUB - 1, _i32(N),
                                     (sid + _i32(1)) * _i32(STRIDE))) - base
            c0 = pltpu.async_copy(l0_hbm.at[pl.ds(base, PER)], l0_v, sem0)
            c1 = pltpu.async_copy(l1_hbm.at[pl.ds(base, PER)], l1_v, sem1)
            c2 = pltpu.async_copy(lab_hbm.at[pl.ds(base, PER)], lab_v, sem2)

            @pl.when(sid == 0)
            def _init():
                counters[_i32(0)] = _i32(0)
                counters[_i32(1)] = _i32(0)
                counters[_i32(2)] = _i32(0)

            c0.wait()
            c1.wait()
            c2.wait()
            plsc.subcore_barrier()

            def body(i, carry):
                offr_v = carry
                for u in range(UNROLL):
                    sl = pl.ds((i * UNROLL + u) * L, L)
                    x0 = l0_v[sl]
                    x1 = l1_v[sl]
                    lb = lab_v[sl]
                    dd = x1 - x0
                    sd = jnp.where(lb == 1, -dd, dd)
                    rel = (lb < 2) & ((lane + (i * UNROLL + u) * L)
                                      < rlim_v)
                    rel_i = jnp.where(rel, _i32(1), _i32(0))
                    pos = offr_v + lax.cumsum(rel_i, axis=0) - _i32(1)
                    plsc.store_scatter(relbuf_d, [pos], sd, mask=rel)
                    plsc.store_scatter(relbuf_f, [pos], lb, mask=rel)
                    offr_v = offr_v + plsc.all_reduce_population_count(rel)
                return offr_v

            offr_v = lax.fori_loop(
                _i32(0), _i32(CH // UNROLL), body, zi)
            offr = jnp.max(offr_v)
            plsc.store_scatter(relbuf_f, [offr + lane], zi + _i32(2))
            crel = lax.shift_right_logical(offr + _i32(L - 1), _i32(4))

            def sp(j, carry):
                off_v, fg_acc, nfg_acc = carry
                sl = pl.ds(j * L, L)
                sd = relbuf_d[sl]
                f = relbuf_f[sl]
                loss = _softplus16(sd)
                is_fg = f == 1
                is_bg = f == 0
                fg_acc = fg_acc + jnp.where(is_fg, loss, _f32(0.0))
                nfg_acc = nfg_acc + jnp.where(is_fg, _i32(1), _i32(0))
                bg_i = jnp.where(is_bg, _i32(1), _i32(0))
                pos = off_v + lax.cumsum(bg_i, axis=0) - _i32(1)
                plsc.store_scatter(bgbuf, [pos], loss, mask=is_bg)
                off_v = off_v + plsc.all_reduce_population_count(is_bg)
                return off_v, fg_acc, nfg_acc

            off_v, fg_acc, nfg_acc = lax.fori_loop(
                _i32(0), crel, sp, (zi, zf, zi))
            off = jnp.max(off_v)
            plsc.store_scatter(bgbuf, [off + lane],
                               jnp.full((L,), -jnp.inf, _f32))

            nch = lax.shift_right_logical(off + _i32(L - 1), _i32(4))
            words = nch * _i32(L)
            gbase = plsc.fetch_and_add(counters.at[_i32(0)], words, subcore_id=_i32(0))
            plsc.fetch_and_add(counters.at[_i32(1)], off, subcore_id=_i32(0))
            nfg_me = jnp.sum(nfg_acc, dtype=_i32)
            plsc.fetch_and_add(counters.at[_i32(2)], nfg_me, subcore_id=_i32(0))

            def cp(j, _):
                pltpu.sync_copy(bgbuf.at[pl.ds(j * L, L)],
                                sh_bg.at[pl.ds(pl.multiple_of(gbase + j * L, L), L)])
                return _

            lax.fori_loop(_i32(0), nch, cp, _i32(0))
            stage_fg[...] = fg_acc
            pltpu.sync_copy(stage_fg, sh_fg.at[pl.ds(sid * L, L)])
            plsc.subcore_barrier()

            @pl.when(sid == 0)
            def _merge():
                gw = counters[_i32(0)]
                n_bg = counters[_i32(1)]
                n_fg = counters[_i32(2)]
                G = lax.shift_right_logical(gw, _i32(4))
                pltpu.sync_copy(sh_fg, meta_fg_v)

                @pl.when(gw <= CAP1)
                def _small():
                    pltpu.sync_copy(sh_bg.at[pl.ds(0, CAP1)],
                                    dense.at[pl.ds(0, CAP1)])

                @pl.when((gw > CAP1) & (gw <= CAP2))
                def _mid():
                    pltpu.sync_copy(sh_bg.at[pl.ds(0, CAP2)],
                                    dense.at[pl.ds(0, CAP2)])

                @pl.when(gw > CAP2)
                def _full():
                    pltpu.sync_copy(sh_bg, dense.at[pl.ds(0, NSUB * SEG)])

                def red(w_, fg_v):
                    return fg_v + meta_fg_v[pl.ds(w_ * L, L)]

                fg_v = lax.fori_loop(_i32(0), _i32(NSUB), red, zf)
                fg_sum = jnp.sum(fg_v)
                K = _i32(NCLS) - n_fg
                ninf = jnp.full((L,), -jnp.inf, _f32)

                def _finish(c_gt, s_gt, tv):
                    t_s = jnp.max(tv)
                    bg_main = s_gt + (K - c_gt).astype(_f32) * t_s
                    bg_sum = jnp.where(
                        K <= _i32(0), _f32(0.0),
                        jnp.where(K > n_bg, _f32(-jnp.inf), bg_main))
                    outbuf[...] = (zf + (fg_sum + bg_sum)) / (zf + _f32(NCLS))
                    pltpu.sync_copy(outbuf, out_hbm)


                @pl.when(G <= _i32(L))
                def _reg_search():
                    vs = [jnp.where(_i32(c) < G, dense[pl.ds(c * L, L)],
                                    ninf) for c in range(L)]

                    def bs(_, carry):
                        lo, hi = carry
                        mid = lo + lax.shift_right_logical(hi - lo, _i32(1))
                        tv = plsc.bitcast(zi + mid, _f32)
                        acc = zi
                        for c in range(L):
                            acc = acc + jnp.where(vs[c] >= tv,
                                                  _i32(1), _i32(0))
                        pred = jnp.sum(acc, dtype=_i32) >= K
                        return (jnp.where(pred, mid, lo),
                                jnp.where(pred, hi, mid))

                    lo, _hi = lax.fori_loop(_i32(0), _i32(31), bs,
                                            (_i32(0), _i32(HI0)))
                    tv = plsc.bitcast(zi + lo, _f32)
                    cv = zi
                    sv = zf
                    for c in range(L):
                        m = vs[c] > tv
                        cv = cv + jnp.where(m, _i32(1), _i32(0))
                        sv = sv + jnp.where(m, vs[c], _f32(0.0))
                    _finish(jnp.sum(cv, dtype=_i32), jnp.sum(sv), tv)

                @pl.when(G > _i32(L))
                def _loop_search():
                    dense[pl.ds(G * L, L)] = ninf
                    G2 = lax.shift_right_logical(G + _i32(1), _i32(1))

                    def bs(_, carry):
                        lo, hi = carry
                        mid = lo + lax.shift_right_logical(hi - lo, _i32(1))
                        tv = plsc.bitcast(zi + mid, _f32)

                        def cb(j, acc):
                            va = dense[pl.ds(j * (2 * L), L)]
                            vb = dense[pl.ds(j * (2 * L) + L, L)]
                            return (acc
                                    + jnp.where(va >= tv, _i32(1), _i32(0))
                                    + jnp.where(vb >= tv, _i32(1), _i32(0)))

                        c = jnp.sum(lax.fori_loop(_i32(0), G2, cb, zi),
                                    dtype=_i32)
                        pred = c >= K
                        return (jnp.where(pred, mid, lo),
                                jnp.where(pred, hi, mid))

                    lo, _hi = lax.fori_loop(_i32(0), _i32(31), bs,
                                            (_i32(0), _i32(HI0)))
                    tv = plsc.bitcast(zi + lo, _f32)

                    def fin(j, carry):
                        cv, sv = carry
                        va = dense[pl.ds(j * (2 * L), L)]
                        vb = dense[pl.ds(j * (2 * L) + L, L)]
                        ma = va > tv
                        mb = vb > tv
                        return (cv + jnp.where(ma, _i32(1), _i32(0))
                                + jnp.where(mb, _i32(1), _i32(0)),
                                sv + jnp.where(ma, va, _f32(0.0))
                                + jnp.where(mb, vb, _f32(0.0)))

                    cv, sv = lax.fori_loop(_i32(0), G2, fin, (zi, zf))
                    _finish(jnp.sum(cv, dtype=_i32), jnp.sum(sv), tv)

    return k


def kernel(输入, 标签):
    logits = 输入[0]
    lab = 标签[0, 0].astype(_i32)
    out = _build()(logits[:, 0], logits[:, 1], lab)
    return out[0]
